# baseline (speedup 1.0000x reference)
import functools

import jax
import jax.numpy as jnp
from jax import lax
from jax.experimental import pallas as pl
from jax.experimental.pallas import tpu as pltpu
from jax.experimental.pallas import tpu_sc as plsc

_N = 10000
_E = 320000
_D = 128
_CHUNK = 64
_NBUF = 4
_NP = 10240
_NC = 2
_NS = 16

_f32 = jnp.float32

_EPAD = 327680
_CPT = (_EPAD // _CHUNK) // (_NC * _NS)
_NSTAGE = 4
_HALF = _CPT // _NSTAGE


def _sc_partials(feat_src, src2d, dst2d):
  mesh = plsc.VectorSubcoreMesh(core_axis_name="c", subcore_axis_name="s")
  rows_per_tile = _NP // _NS

  @functools.partial(
      pl.kernel,
      mesh=mesh,
      out_type=(
          jax.ShapeDtypeStruct((_NC, _NP, _D), _f32),
          jax.ShapeDtypeStruct((_NC, _NP), _f32),
      ),
      scratch_types=[
          pltpu.VMEM((_CHUNK, _D), _f32),
          pltpu.VMEM((_CHUNK, _D), _f32),
          pltpu.VMEM((_CHUNK, _D), _f32),
          pltpu.VMEM((_CHUNK, _D), _f32),
          pltpu.VMEM((_HALF, _CHUNK), jnp.int32),
          pltpu.VMEM((_HALF, _CHUNK), jnp.int32),
          pltpu.VMEM((_CHUNK,), _f32),
          pltpu.VMEM((rows_per_tile,), _f32),
          pltpu.VMEM_SHARED((_NP, _D), _f32),
          pltpu.VMEM_SHARED((_NP,), _f32),
          pltpu.SemaphoreType.DMA,
          pltpu.SemaphoreType.DMA,
          pltpu.SemaphoreType.DMA,
          pltpu.SemaphoreType.DMA,
      ],
  )
  def body(feat_hbm, src_hbm, dst_hbm, zeros_hbm, psum_out, pdeg_out,
           rbuf0, rbuf1, rbuf2, rbuf3, sbuf, dbuf, ones, zdeg, acc, deg,
           gsem0, gsem1, gsem2, gsem3):
    c = lax.axis_index("c")
    s = lax.axis_index("s")
    t = c * _NS + s

    for i in range(_CHUNK // 16):
      ones[pl.ds(16 * i, 16)] = jnp.full((16,), 1.0, _f32)
    for i in range(rows_per_tile // 16):
      zdeg[pl.ds(16 * i, 16)] = jnp.zeros((16,), _f32)

    base = s * rows_per_tile
    pltpu.sync_copy(zeros_hbm.at[pl.ds(base, rows_per_tile), :],
                    acc.at[pl.ds(base, rows_per_tile), :])
    pltpu.sync_copy(zdeg, deg.at[pl.ds(base, rows_per_tile)])
    plsc.subcore_barrier()

    rbufs = (rbuf0, rbuf1, rbuf2, rbuf3)
    gsems = (gsem0, gsem1, gsem2, gsem3)
    c0 = t * _CPT

    for h in range(_NSTAGE):
      pltpu.sync_copy(src_hbm.at[pl.ds(c0 + h * _HALF, _HALF)], sbuf)
      pltpu.sync_copy(dst_hbm.at[pl.ds(c0 + h * _HALF, _HALF)], dbuf)

      for b in range(_NBUF):
        pltpu.async_copy(feat_hbm.at[sbuf.at[b]], rbufs[b], gsems[b])

      def group(p, carry):
        for b in range(_NBUF):
          j = _NBUF * p + b
          rb, gs = rbufs[b], gsems[b]
          pltpu.make_async_copy(feat_hbm.at[pl.ds(0, _CHUNK), :], rb, gs).wait()
          pltpu.sync_copy(rb, acc.at[dbuf.at[j]], add=True)
          pltpu.sync_copy(ones, deg.at[dbuf.at[j]], add=True)
          @pl.when(j + _NBUF < _HALF)
          def _():
            pltpu.async_copy(feat_hbm.at[sbuf.at[j + _NBUF]], rb, gs)
        return carry

      lax.fori_loop(0, _HALF // _NBUF, group, 0)

    plsc.subcore_barrier()

    pltpu.sync_copy(acc.at[pl.ds(base, rows_per_tile), :],
                    psum_out.at[c, pl.ds(base, rows_per_tile), :])
    pltpu.sync_copy(deg.at[pl.ds(base, rows_per_tile)],
                    pdeg_out.at[c, pl.ds(base, rows_per_tile)])

  return body(feat_src, src2d, dst2d, jnp.zeros((_NP, _D), _f32))


def _combine(psum, pdeg, h_self):
  p0, p1 = psum[0], psum[1]
  d0 = pdeg[0].reshape(-1, 1)
  d1 = pdeg[1].reshape(-1, 1)
  rows = 1000
  grid = (_N // rows,)

  def body(p0_ref, p1_ref, d0_ref, d1_ref, h_ref, o_ref):
    degree = jnp.maximum(d0_ref[...] + d1_ref[...], 1.0)
    o_ref[...] = jnp.maximum(
        (p0_ref[...] + p1_ref[...]) / degree + h_ref[...], 0.0)

  return pl.pallas_call(
      body,
      grid=grid,
      in_specs=[
          pl.BlockSpec((rows, _D), lambda i: (i, 0)),
          pl.BlockSpec((rows, _D), lambda i: (i, 0)),
          pl.BlockSpec((rows, 1), lambda i: (i, 0)),
          pl.BlockSpec((rows, 1), lambda i: (i, 0)),
          pl.BlockSpec((rows, _D), lambda i: (i, 0)),
      ],
      out_specs=pl.BlockSpec((rows, _D), lambda i: (i, 0)),
      out_shape=jax.ShapeDtypeStruct((_N, _D), _f32),
  )(p0, p1, d0, d1, h_self)


def kernel(feat_src, h_self, edge_index):
  npad = _EPAD - _E
  src2d = jnp.concatenate(
      [edge_index[0], jnp.zeros((npad,), jnp.int32)]).reshape(-1, _CHUNK)
  dst2d = jnp.concatenate(
      [edge_index[1], jnp.full((npad,), _NP - 1, jnp.int32)]).reshape(-1, _CHUNK)
  psum, pdeg = _sc_partials(feat_src, src2d, dst2d)
  return _combine(psum, pdeg, h_self)


# speedup vs baseline: 1.0163x; 1.0163x over previous
import functools

import jax
import jax.numpy as jnp
from jax import lax
from jax.experimental import pallas as pl
from jax.experimental.pallas import tpu as pltpu
from jax.experimental.pallas import tpu_sc as plsc

_N = 10000
_E = 320000
_D = 128
_CHUNK = 128
_NP = 10240
_NC = 2
_NS = 16

_f32 = jnp.float32

_EPAD = 2560 * _CHUNK
_CPT = (_EPAD // _CHUNK) // (_NC * _NS)
_HALF = _CPT // 2


def _sc_partials(feat_src, src2d, dst2d):
  mesh = plsc.VectorSubcoreMesh(core_axis_name="c", subcore_axis_name="s")
  rows_per_tile = _NP // _NS

  @functools.partial(
      pl.kernel,
      mesh=mesh,
      out_type=(
          jax.ShapeDtypeStruct((_NC, _NP, _D), _f32),
          jax.ShapeDtypeStruct((_NC, _NP), _f32),
      ),
      scratch_types=[
          pltpu.VMEM((_CHUNK, _D), _f32),
          pltpu.VMEM((_CHUNK, _D), _f32),
          pltpu.VMEM((_HALF, _CHUNK), jnp.int32),
          pltpu.VMEM((_HALF, _CHUNK), jnp.int32),
          pltpu.VMEM((_CHUNK,), _f32),
          pltpu.VMEM((rows_per_tile,), _f32),
          pltpu.VMEM_SHARED((_NP, _D), _f32),
          pltpu.VMEM_SHARED((_NP,), _f32),
          pltpu.SemaphoreType.DMA,
          pltpu.SemaphoreType.DMA,
      ],
  )
  def body(feat_hbm, src_hbm, dst_hbm, zeros_hbm, psum_out, pdeg_out,
           rbuf0, rbuf1, sbuf, dbuf, ones, zdeg, acc, deg, gsem0, gsem1):
    c = lax.axis_index("c")
    s = lax.axis_index("s")
    t = c * _NS + s

    for i in range(_CHUNK // 16):
      ones[pl.ds(16 * i, 16)] = jnp.full((16,), 1.0, _f32)
    for i in range(rows_per_tile // 16):
      zdeg[pl.ds(16 * i, 16)] = jnp.zeros((16,), _f32)

    base = s * rows_per_tile
    pltpu.sync_copy(zeros_hbm.at[pl.ds(base, rows_per_tile), :],
                    acc.at[pl.ds(base, rows_per_tile), :])
    pltpu.sync_copy(zdeg, deg.at[pl.ds(base, rows_per_tile)])
    plsc.subcore_barrier()

    rbufs = (rbuf0, rbuf1)
    gsems = (gsem0, gsem1)
    c0 = t * _CPT

    for h in range(2):
      pltpu.sync_copy(src_hbm.at[pl.ds(c0 + h * _HALF, _HALF)], sbuf)
      pltpu.sync_copy(dst_hbm.at[pl.ds(c0 + h * _HALF, _HALF)], dbuf)

      for b in range(2):
        pltpu.async_copy(feat_hbm.at[sbuf.at[b]], rbufs[b], gsems[b])

      def pair(p, carry):
        for b in range(2):
          j = 2 * p + b
          rb, gs = rbufs[b], gsems[b]
          pltpu.make_async_copy(feat_hbm.at[pl.ds(0, _CHUNK), :], rb, gs).wait()
          pltpu.sync_copy(rb, acc.at[dbuf.at[j]], add=True)
          pltpu.sync_copy(ones, deg.at[dbuf.at[j]], add=True)
          @pl.when(j + 2 < _HALF)
          def _():
            pltpu.async_copy(feat_hbm.at[sbuf.at[j + 2]], rb, gs)
        return carry

      lax.fori_loop(0, _HALF // 2, pair, 0)

    plsc.subcore_barrier()

    pltpu.sync_copy(acc.at[pl.ds(base, rows_per_tile), :],
                    psum_out.at[c, pl.ds(base, rows_per_tile), :])
    pltpu.sync_copy(deg.at[pl.ds(base, rows_per_tile)],
                    pdeg_out.at[c, pl.ds(base, rows_per_tile)])

  return body(feat_src, src2d, dst2d, jnp.zeros((_NP, _D), _f32))


def _combine(psum, pdeg, h_self):
  p0, p1 = psum[0], psum[1]
  d0 = pdeg[0].reshape(-1, 1)
  d1 = pdeg[1].reshape(-1, 1)
  rows = 1000
  grid = (_N // rows,)

  def body(p0_ref, p1_ref, d0_ref, d1_ref, h_ref, o_ref):
    degree = jnp.maximum(d0_ref[...] + d1_ref[...], 1.0)
    o_ref[...] = jnp.maximum(
        (p0_ref[...] + p1_ref[...]) / degree + h_ref[...], 0.0)

  return pl.pallas_call(
      body,
      grid=grid,
      in_specs=[
          pl.BlockSpec((rows, _D), lambda i: (i, 0)),
          pl.BlockSpec((rows, _D), lambda i: (i, 0)),
          pl.BlockSpec((rows, 1), lambda i: (i, 0)),
          pl.BlockSpec((rows, 1), lambda i: (i, 0)),
          pl.BlockSpec((rows, _D), lambda i: (i, 0)),
      ],
      out_specs=pl.BlockSpec((rows, _D), lambda i: (i, 0)),
      out_shape=jax.ShapeDtypeStruct((_N, _D), _f32),
  )(p0, p1, d0, d1, h_self)


def kernel(feat_src, h_self, edge_index):
  npad = _EPAD - _E
  src2d = jnp.concatenate(
      [edge_index[0], jnp.zeros((npad,), jnp.int32)]).reshape(-1, _CHUNK)
  dst2d = jnp.concatenate(
      [edge_index[1], jnp.full((npad,), _NP - 1, jnp.int32)]).reshape(-1, _CHUNK)
  psum, pdeg = _sc_partials(feat_src, src2d, dst2d)
  return _combine(psum, pdeg, h_self)
